# Initial kernel scaffold; baseline (speedup 1.0000x reference)
import jax
import jax.numpy as jnp
from jax.experimental import pallas as pl


def kernel(x, Wqkv, bqkv, Wproj, bproj):
    raise NotImplementedError("write your pallas kernel here")


# speedup vs baseline: 1.8027x; 1.8027x over previous
import functools
import math

import jax
import jax.numpy as jnp
from jax import lax
from jax.experimental import pallas as pl
from jax.experimental.pallas import tpu as pltpu
from jax.experimental.pallas import tpu_sc as plsc

_B, _T, _D = 2, 8192, 768
_H = 12
_HD = _D // _H
_BUCKET = 64
_NBUCKETS = _T // _BUCKET
_CHUNK = 2 * _BUCKET
_NCHUNKS = _T // _CHUNK
_SEED = 1234

_ROWBLK = 512
_GIN_CH = 128


def _rope_tables():
    inv_freq = 1.0 / (10000.0 ** (jnp.arange(0, _HD, 2, dtype=jnp.float32) / _HD))
    t = jnp.arange(_T, dtype=jnp.float32)
    freqs = jnp.einsum('i,j->ij', t, inv_freq)
    emb = jnp.concatenate([freqs, freqs], axis=-1)
    return jnp.cos(emb), jnp.sin(emb)


def _qkv_rope_hash_body(x_ref, w_ref, b_ref, cos_ref, sin_ref, hw_ref,
                        out_ref, bidx_ref):
    acc = jnp.dot(x_ref[...], w_ref[...],
                  preferred_element_type=jnp.float32) + b_ref[...]
    cos = cos_ref[...]
    sin = sin_ref[...]
    ones = jnp.ones_like(cos)
    zeros = jnp.zeros_like(sin)
    cos_f = jnp.concatenate([cos, cos, ones] * _H, axis=1)
    sin_f = jnp.concatenate([sin, sin, zeros] * _H, axis=1)
    r_left = jnp.roll(acc, -1, axis=1)
    r_right = jnp.roll(acc, 1, axis=1)
    lane = lax.broadcasted_iota(jnp.int32, acc.shape, 1)
    rot = jnp.where(lane % 2 == 0, -r_left, r_right)
    roped = acc * cos_f + rot * sin_f
    pad = jnp.zeros((acc.shape[0], _HD), jnp.float32)
    pieces = []
    for h in range(_H):
        pieces.append(roped[:, h * 192:(h + 1) * 192])
        pieces.append(pad)
    out_ref[...] = jnp.concatenate(pieces, axis=1)
    qcat = jnp.concatenate(
        [roped[:, h * 192:h * 192 + _HD] for h in range(_H)], axis=1)
    p = jnp.dot(qcat, hw_ref[...], preferred_element_type=jnp.float32)
    cols = []
    for h in range(_H):
        ph = p[:, h * _HD:(h + 1) * _HD]
        full = jnp.concatenate([ph, -ph], axis=1)
        cols.append(jnp.argmax(full, axis=1, keepdims=True).astype(jnp.int32))
    bidx_ref[...] = jnp.concatenate(cols, axis=1)


def _qkv_rope_hash(x2, w_perm, b_perm, cos, sin, hash_w):
    n_rows = x2.shape[0]
    grid = (n_rows // _ROWBLK,)
    return pl.pallas_call(
        _qkv_rope_hash_body,
        grid=grid,
        in_specs=[
            pl.BlockSpec((_ROWBLK, _D), lambda i: (i, 0)),
            pl.BlockSpec((_D, 3 * _D), lambda i: (0, 0)),
            pl.BlockSpec((1, 3 * _D), lambda i: (0, 0)),
            pl.BlockSpec((_ROWBLK, _HD), lambda i: (i % (_T // _ROWBLK), 0)),
            pl.BlockSpec((_ROWBLK, _HD), lambda i: (i % (_T // _ROWBLK), 0)),
            pl.BlockSpec((_D, _D), lambda i: (0, 0)),
        ],
        out_specs=[
            pl.BlockSpec((_ROWBLK, _H * 256), lambda i: (i, 0)),
            pl.BlockSpec((_ROWBLK, _H), lambda i: (i, 0)),
        ],
        out_shape=[
            jax.ShapeDtypeStruct((n_rows, _H * 256), jnp.float32),
            jax.ShapeDtypeStruct((n_rows, _H), jnp.int32),
        ],
    )(x2, w_perm, b_perm, cos, sin, hash_w)


def _chunk_attn_body(s_ref, o_ref):
    s = s_ref[0]
    q = s[:, :_HD]
    k = s[:, _HD:2 * _HD]
    v = s[:, 2 * _HD:3 * _HD]
    a = lax.dot_general(q, k, (((1,), (1,)), ((), ())),
                        preferred_element_type=jnp.float32)
    a = a * (1.0 / math.sqrt(_HD))
    m = jnp.max(a, axis=-1, keepdims=True)
    e = jnp.exp(a - m)
    a = e / jnp.sum(e, axis=-1, keepdims=True)
    o = jnp.dot(a, v, preferred_element_type=jnp.float32)
    o_ref[0] = jnp.concatenate(
        [o, jnp.zeros((_CHUNK, _HD), jnp.float32)], axis=1)


def _chunk_attn(sorted_rows):
    n_blk = sorted_rows.shape[0]
    return pl.pallas_call(
        _chunk_attn_body,
        grid=(n_blk,),
        in_specs=[pl.BlockSpec((1, _CHUNK, 4 * _HD), lambda i: (i, 0, 0))],
        out_specs=pl.BlockSpec((1, _CHUNK, 2 * _HD), lambda i: (i, 0, 0)),
        out_shape=jax.ShapeDtypeStruct((n_blk, _CHUNK, 2 * _HD), jnp.float32),
    )(sorted_rows)


def _proj_body(x_ref, w_ref, b_ref, o_ref):
    o_ref[...] = jnp.dot(x_ref[...], w_ref[...],
                         preferred_element_type=jnp.float32) + b_ref[...]


def _proj(y, w, b2):
    n_rows = y.shape[0]
    d_in = y.shape[1]
    return pl.pallas_call(
        _proj_body,
        grid=(n_rows // _ROWBLK,),
        in_specs=[
            pl.BlockSpec((_ROWBLK, d_in), lambda i: (i, 0)),
            pl.BlockSpec((d_in, _D), lambda i: (0, 0)),
            pl.BlockSpec((1, _D), lambda i: (0, 0)),
        ],
        out_specs=pl.BlockSpec((_ROWBLK, _D), lambda i: (i, 0)),
        out_shape=jax.ShapeDtypeStruct((n_rows, _D), jnp.float32),
    )(y, w, b2)


def _gather_rows(table, idx, width):
    n_rows = idx.shape[0]
    info = plsc.get_sparse_core_info()
    nc, ns = info.num_cores, info.num_subcores
    nw = nc * ns
    rows_per_w = n_rows // nw
    n_chunk = rows_per_w // _GIN_CH
    idx3 = idx.reshape(nw, n_chunk, _GIN_CH)
    mesh = plsc.VectorSubcoreMesh(core_axis_name="c", subcore_axis_name="s")

    @functools.partial(
        pl.kernel, mesh=mesh,
        out_type=jax.ShapeDtypeStruct((n_rows, width), jnp.float32),
        scratch_types=[
            pltpu.VMEM((n_chunk, _GIN_CH), jnp.int32),
            pltpu.VMEM((_GIN_CH, width), jnp.float32),
            pltpu.SemaphoreType.DMA,
        ],
    )
    def k(table_hbm, idx_hbm, out_hbm, idx_v, rows_v, sem):
        wid = lax.axis_index("s") * nc + lax.axis_index("c")
        base = wid * rows_per_w
        pltpu.sync_copy(idx_hbm.at[wid], idx_v)

        def body(j, carry):
            pltpu.async_copy(table_hbm.at[idx_v.at[j]], rows_v, sem).wait()
            pltpu.sync_copy(rows_v,
                            out_hbm.at[pl.ds(base + j * _GIN_CH, _GIN_CH)])
            return carry

        lax.fori_loop(0, n_chunk, body, 0)

    return k(table, idx3)


def kernel(x, Wqkv, bqkv, Wproj, bproj):
    h_idx = jnp.arange(3 * _D)
    h = h_idx // 192
    g = (h_idx % 192) // _HD
    d = h_idx % _HD
    perm = g * _D + h * _HD + d
    w_perm = Wqkv[:, perm]
    b_perm = bqkv[perm].reshape(1, 3 * _D)

    cos, sin = _rope_tables()
    R = jax.random.normal(jax.random.key(_SEED), (_HD, 2, _NBUCKETS // 2),
                          dtype=x.dtype)
    hash_w = jnp.kron(jnp.eye(_H, dtype=jnp.float32), R[:, 0, :])

    x2 = x.reshape(_B * _T, _D)

    qkv_rows, bidx = _qkv_rope_hash(x2, w_perm, b_perm, cos, sin, hash_w)
    table = qkv_rows.reshape(_B * _T * _H, 4 * _HD)

    bidx3 = bidx.reshape(_B, _T, _H).transpose(0, 2, 1).reshape(_B * _H, _T)
    sorted_idx = jnp.argsort(bidx3, axis=-1).astype(jnp.int32)
    bb = (jnp.arange(_B * _H, dtype=jnp.int32) // _H)[:, None]
    hh = (jnp.arange(_B * _H, dtype=jnp.int32) % _H)[:, None]
    gidx_in = ((bb * _T + sorted_idx) * _H + hh).reshape(-1)

    pos = jnp.argsort(sorted_idx, axis=-1).astype(jnp.int32)
    src = jnp.arange(_B * _H, dtype=jnp.int32)[:, None] * _T + pos
    gidx_out = (src.reshape(_B, _H, _T).transpose(0, 2, 1)).reshape(-1)

    sorted_rows = _gather_rows(table, gidx_in, 4 * _HD)
    sorted_blk = sorted_rows.reshape(_B * _H * _NCHUNKS, _CHUNK, 4 * _HD)

    out_local = _chunk_attn(sorted_blk).reshape(_B * _H * _T, 2 * _HD)

    out_rows = _gather_rows(out_local, gidx_out, 2 * _HD)
    y = out_rows.reshape(_B * _T, 2 * _D)

    w2 = jnp.zeros((_H, 2 * _HD, _D), jnp.float32)
    w2 = w2.at[:, :_HD, :].set(Wproj.reshape(_H, _HD, _D))
    out = _proj(y, w2.reshape(2 * _D, _D), bproj.reshape(1, _D))
    return out.reshape(_B, _T, _D)


# speedup vs baseline: 1.8209x; 1.0101x over previous
import functools
import math

import jax
import jax.numpy as jnp
from jax import lax
from jax.experimental import pallas as pl
from jax.experimental.pallas import tpu as pltpu
from jax.experimental.pallas import tpu_sc as plsc

_B, _T, _D = 2, 8192, 768
_H = 12
_HD = _D // _H
_BUCKET = 64
_NBUCKETS = _T // _BUCKET
_CHUNK = 2 * _BUCKET
_NCHUNKS = _T // _CHUNK
_SEED = 1234

_ROWBLK = 512
_GIN_CH = 128


def _rope_tables():
    inv_freq = 1.0 / (10000.0 ** (jnp.arange(0, _HD, 2, dtype=jnp.float32) / _HD))
    t = jnp.arange(_T, dtype=jnp.float32)
    freqs = jnp.einsum('i,j->ij', t, inv_freq)
    emb = jnp.concatenate([freqs, freqs], axis=-1)
    return jnp.cos(emb), jnp.sin(emb)


def _qkv_rope_hash_body(x_ref, w_ref, b_ref, cos_ref, sin_ref, hw_ref,
                        out_ref, bidx_ref):
    acc = jnp.dot(x_ref[...], w_ref[...],
                  preferred_element_type=jnp.float32) + b_ref[...]
    cos = cos_ref[...]
    sin = sin_ref[...]
    ones = jnp.ones_like(cos)
    zeros = jnp.zeros_like(sin)
    cos_f = jnp.concatenate([cos, cos, ones] * _H, axis=1)
    sin_f = jnp.concatenate([sin, sin, zeros] * _H, axis=1)
    r_left = jnp.roll(acc, -1, axis=1)
    r_right = jnp.roll(acc, 1, axis=1)
    lane = lax.broadcasted_iota(jnp.int32, acc.shape, 1)
    rot = jnp.where(lane % 2 == 0, -r_left, r_right)
    roped = acc * cos_f + rot * sin_f
    pad = jnp.zeros((acc.shape[0], _HD), jnp.float32)
    pieces = []
    for h in range(_H):
        pieces.append(roped[:, h * 192:(h + 1) * 192])
        pieces.append(pad)
    out_ref[...] = jnp.concatenate(pieces, axis=1)
    r0 = hw_ref[...]
    cols = []
    for h in range(_H):
        ph = jnp.dot(roped[:, h * 192:h * 192 + _HD], r0,
                     preferred_element_type=jnp.float32)
        full = jnp.concatenate([ph, -ph], axis=1)
        cols.append(jnp.argmax(full, axis=1, keepdims=True).astype(jnp.int32))
    bidx_ref[...] = jnp.concatenate(cols, axis=1)


def _qkv_rope_hash(x2, w_perm, b_perm, cos, sin, hash_w):
    n_rows = x2.shape[0]
    grid = (n_rows // _ROWBLK,)
    return pl.pallas_call(
        _qkv_rope_hash_body,
        grid=grid,
        in_specs=[
            pl.BlockSpec((_ROWBLK, _D), lambda i: (i, 0)),
            pl.BlockSpec((_D, 3 * _D), lambda i: (0, 0)),
            pl.BlockSpec((1, 3 * _D), lambda i: (0, 0)),
            pl.BlockSpec((_ROWBLK, _HD), lambda i: (i % (_T // _ROWBLK), 0)),
            pl.BlockSpec((_ROWBLK, _HD), lambda i: (i % (_T // _ROWBLK), 0)),
            pl.BlockSpec((_HD, _HD), lambda i: (0, 0)),
        ],
        out_specs=[
            pl.BlockSpec((_ROWBLK, _H * 256), lambda i: (i, 0)),
            pl.BlockSpec((_ROWBLK, _H), lambda i: (i, 0)),
        ],
        out_shape=[
            jax.ShapeDtypeStruct((n_rows, _H * 256), jnp.float32),
            jax.ShapeDtypeStruct((n_rows, _H), jnp.int32),
        ],
    )(x2, w_perm, b_perm, cos, sin, hash_w)


_LPC = 128
_NTC = _T // _LPC


def _count_sort_body(b_ref, tu128_ref, tu64_ref, tl128_ref, dest_ref):
    br = b_ref[0]
    bins = lax.broadcasted_iota(jnp.int32, (_NBUCKETS, _NTC, _LPC), 0)
    m3 = (br[None, :, :] == bins).astype(jnp.float32)
    r3 = lax.dot_general(m3, tu128_ref[...], (((2,), (0,)), ((), ())),
                         preferred_element_type=jnp.float32)
    g = jnp.sum(m3, axis=2)
    e = jnp.dot(g, tu64_ref[...],
                preferred_element_type=jnp.float32)
    cnt = jnp.sum(g, axis=1, keepdims=True)
    bo = jnp.dot(tl128_ref[...], cnt,
                 preferred_element_type=jnp.float32)
    dest3 = m3 * (r3 + e[:, :, None] + bo[:, :, None])
    dest_ref[0] = jnp.sum(dest3, axis=0).astype(jnp.int32)


def _count_sort(bidx3d):
    i128 = jnp.arange(_NBUCKETS)
    tu128 = (i128[:, None] < i128[None, :]).astype(jnp.float32)
    i64 = jnp.arange(_NTC)
    tu64 = (i64[:, None] < i64[None, :]).astype(jnp.float32)
    tl128 = (i128[None, :] < i128[:, None]).astype(jnp.float32)
    return pl.pallas_call(
        _count_sort_body,
        grid=(_B * _H,),
        in_specs=[
            pl.BlockSpec((1, _NTC, _LPC), lambda i: (i, 0, 0)),
            pl.BlockSpec((_NBUCKETS, _LPC), lambda i: (0, 0)),
            pl.BlockSpec((_NTC, _NTC), lambda i: (0, 0)),
            pl.BlockSpec((_NBUCKETS, _NBUCKETS), lambda i: (0, 0)),
        ],
        out_specs=pl.BlockSpec((1, _NTC, _LPC), lambda i: (i, 0, 0)),
        out_shape=jax.ShapeDtypeStruct((_B * _H, _NTC, _LPC), jnp.int32),
    )(bidx3d, tu128, tu64, tl128)


def _chunk_attn_body(s_ref, o_ref):
    s = s_ref[0]
    q = s[:, :_HD]
    k = s[:, _HD:2 * _HD]
    v = s[:, 2 * _HD:3 * _HD]
    a = lax.dot_general(q, k, (((1,), (1,)), ((), ())),
                        preferred_element_type=jnp.float32)
    a = a * (1.0 / math.sqrt(_HD))
    m = jnp.max(a, axis=-1, keepdims=True)
    e = jnp.exp(a - m)
    a = e / jnp.sum(e, axis=-1, keepdims=True)
    o = jnp.dot(a, v, preferred_element_type=jnp.float32)
    o_ref[0] = jnp.concatenate(
        [o, jnp.zeros((_CHUNK, _HD), jnp.float32)], axis=1)


def _chunk_attn(sorted_rows):
    n_blk = sorted_rows.shape[0]
    return pl.pallas_call(
        _chunk_attn_body,
        grid=(n_blk,),
        in_specs=[pl.BlockSpec((1, _CHUNK, 4 * _HD), lambda i: (i, 0, 0))],
        out_specs=pl.BlockSpec((1, _CHUNK, 2 * _HD), lambda i: (i, 0, 0)),
        out_shape=jax.ShapeDtypeStruct((n_blk, _CHUNK, 2 * _HD), jnp.float32),
    )(sorted_rows)


def _proj_body(x_ref, w_ref, b_ref, o_ref):
    o_ref[...] = jnp.dot(x_ref[...], w_ref[...],
                         preferred_element_type=jnp.float32) + b_ref[...]


def _proj(y, w, b2):
    n_rows = y.shape[0]
    d_in = y.shape[1]
    return pl.pallas_call(
        _proj_body,
        grid=(n_rows // _ROWBLK,),
        in_specs=[
            pl.BlockSpec((_ROWBLK, d_in), lambda i: (i, 0)),
            pl.BlockSpec((d_in, _D), lambda i: (0, 0)),
            pl.BlockSpec((1, _D), lambda i: (0, 0)),
        ],
        out_specs=pl.BlockSpec((_ROWBLK, _D), lambda i: (i, 0)),
        out_shape=jax.ShapeDtypeStruct((n_rows, _D), jnp.float32),
    )(y, w, b2)


def _gather_rows(table, idx, width):
    n_rows = idx.shape[0]
    info = plsc.get_sparse_core_info()
    nc, ns = info.num_cores, info.num_subcores
    nw = nc * ns
    rows_per_w = n_rows // nw
    n_chunk = rows_per_w // _GIN_CH
    idx3 = idx.reshape(nw, n_chunk, _GIN_CH)
    mesh = plsc.VectorSubcoreMesh(core_axis_name="c", subcore_axis_name="s")

    @functools.partial(
        pl.kernel, mesh=mesh,
        out_type=jax.ShapeDtypeStruct((n_rows, width), jnp.float32),
        scratch_types=[
            pltpu.VMEM((n_chunk, _GIN_CH), jnp.int32),
            pltpu.VMEM((_GIN_CH, width), jnp.float32),
            pltpu.SemaphoreType.DMA,
        ],
    )
    def k(table_hbm, idx_hbm, out_hbm, idx_v, rows_v, sem):
        wid = lax.axis_index("s") * nc + lax.axis_index("c")
        base = wid * rows_per_w
        pltpu.sync_copy(idx_hbm.at[wid], idx_v)

        def body(j, carry):
            pltpu.async_copy(table_hbm.at[idx_v.at[j]], rows_v, sem).wait()
            pltpu.sync_copy(rows_v,
                            out_hbm.at[pl.ds(base + j * _GIN_CH, _GIN_CH)])
            return carry

        lax.fori_loop(0, n_chunk, body, 0)

    return k(table, idx3)


def _scatter_rows(table, idx, width):
    n_rows = idx.shape[0]
    info = plsc.get_sparse_core_info()
    nc, ns = info.num_cores, info.num_subcores
    nw = nc * ns
    rows_per_w = n_rows // nw
    n_chunk = rows_per_w // _GIN_CH
    idx3 = idx.reshape(nw, n_chunk, _GIN_CH)
    mesh = plsc.VectorSubcoreMesh(core_axis_name="c", subcore_axis_name="s")

    @functools.partial(
        pl.kernel, mesh=mesh,
        out_type=jax.ShapeDtypeStruct((n_rows, width), jnp.float32),
        scratch_types=[
            pltpu.VMEM((n_chunk, _GIN_CH), jnp.int32),
            pltpu.VMEM((_GIN_CH, width), jnp.float32),
            pltpu.SemaphoreType.DMA,
        ],
    )
    def k(table_hbm, idx_hbm, out_hbm, idx_v, rows_v, sem):
        wid = lax.axis_index("s") * nc + lax.axis_index("c")
        base = wid * rows_per_w
        pltpu.sync_copy(idx_hbm.at[wid], idx_v)

        def body(j, carry):
            pltpu.sync_copy(table_hbm.at[pl.ds(base + j * _GIN_CH, _GIN_CH)],
                            rows_v)
            pltpu.async_copy(rows_v, out_hbm.at[idx_v.at[j]], sem).wait()
            return carry

        lax.fori_loop(0, n_chunk, body, 0)

    return k(table, idx3)


def kernel(x, Wqkv, bqkv, Wproj, bproj):
    h_idx = jnp.arange(3 * _D)
    h = h_idx // 192
    g = (h_idx % 192) // _HD
    d = h_idx % _HD
    perm = g * _D + h * _HD + d
    w_perm = Wqkv[:, perm]
    b_perm = bqkv[perm].reshape(1, 3 * _D)

    cos, sin = _rope_tables()
    R = jax.random.normal(jax.random.key(_SEED), (_HD, 2, _NBUCKETS // 2),
                          dtype=x.dtype)
    r0 = R[:, 0, :]

    x2 = x.reshape(_B * _T, _D)

    qkv_rows, bidx = _qkv_rope_hash(x2, w_perm, b_perm, cos, sin, r0)
    table = qkv_rows.reshape(_B * _T * _H, 4 * _HD)

    bidx3d = bidx.reshape(_B, _T, _H).transpose(0, 2, 1).reshape(
        _B * _H, _NTC, _LPC)
    dest = _count_sort(bidx3d).reshape(_B * _H, _T)
    src = jnp.arange(_B * _H, dtype=jnp.int32)[:, None] * _T + dest
    gidx = (src.reshape(_B, _H, _T).transpose(0, 2, 1)).reshape(-1)

    sorted_rows = _scatter_rows(table, gidx, 4 * _HD)
    sorted_blk = sorted_rows.reshape(_B * _H * _NCHUNKS, _CHUNK, 4 * _HD)

    out_local = _chunk_attn(sorted_blk).reshape(_B * _H * _T, 2 * _HD)

    out_rows = _gather_rows(out_local, gidx, 2 * _HD)
    y = out_rows.reshape(_B * _T, 2 * _D)

    w2 = jnp.zeros((_H, 2 * _HD, _D), jnp.float32)
    w2 = w2.at[:, :_HD, :].set(Wproj.reshape(_H, _HD, _D))
    out = _proj(y, w2.reshape(2 * _D, _D), bproj.reshape(1, _D))
    return out.reshape(_B, _T, _D)


# speedup vs baseline: 3.3851x; 1.8590x over previous
import functools
import math

import jax
import jax.numpy as jnp
from jax import lax
from jax.experimental import pallas as pl
from jax.experimental.pallas import tpu as pltpu
from jax.experimental.pallas import tpu_sc as plsc

_B, _T, _D = 2, 8192, 768
_H = 12
_HD = _D // _H
_BUCKET = 64
_NBUCKETS = _T // _BUCKET
_CHUNK = 2 * _BUCKET
_NCHUNKS = _T // _CHUNK
_SEED = 1234

_ROWBLK = 512
_GIN_CH = 128


def _rope_tables():
    inv_freq = 1.0 / (10000.0 ** (jnp.arange(0, _HD, 2, dtype=jnp.float32) / _HD))
    t = jnp.arange(_T, dtype=jnp.float32)
    freqs = jnp.einsum('i,j->ij', t, inv_freq)
    emb = jnp.concatenate([freqs, freqs], axis=-1)
    return jnp.cos(emb), jnp.sin(emb)


def _qkv_rope_hash_body(x_ref, w_ref, b_ref, cos_ref, sin_ref, hw_ref,
                        out_ref, bidx_ref):
    acc = jnp.dot(x_ref[...], w_ref[...],
                  preferred_element_type=jnp.float32) + b_ref[...]
    cos = cos_ref[...]
    sin = sin_ref[...]
    ones = jnp.ones_like(cos)
    zeros = jnp.zeros_like(sin)
    cos_f = jnp.concatenate([cos, cos, ones] * _H, axis=1)
    sin_f = jnp.concatenate([sin, sin, zeros] * _H, axis=1)
    r_left = jnp.roll(acc, -1, axis=1)
    r_right = jnp.roll(acc, 1, axis=1)
    lane = lax.broadcasted_iota(jnp.int32, acc.shape, 1)
    rot = jnp.where(lane % 2 == 0, -r_left, r_right)
    roped = acc * cos_f + rot * sin_f
    pad = jnp.zeros((acc.shape[0], _HD), jnp.float32)
    pieces = []
    for h in range(_H):
        pieces.append(roped[:, h * 192:(h + 1) * 192])
        pieces.append(pad)
    out_ref[...] = jnp.concatenate(pieces, axis=1)
    r0 = hw_ref[...]
    cols = []
    for h in range(_H):
        ph = jnp.dot(roped[:, h * 192:h * 192 + _HD], r0,
                     preferred_element_type=jnp.float32)
        full = jnp.concatenate([ph, -ph], axis=1)
        cols.append(jnp.argmax(full, axis=1, keepdims=True).astype(jnp.int32))
    bidx_ref[...] = jnp.concatenate(cols, axis=1)


def _qkv_rope_hash(x2, w_perm, b_perm, cos, sin, hash_w):
    n_rows = x2.shape[0]
    grid = (n_rows // _ROWBLK,)
    return pl.pallas_call(
        _qkv_rope_hash_body,
        grid=grid,
        in_specs=[
            pl.BlockSpec((_ROWBLK, _D), lambda i: (i, 0)),
            pl.BlockSpec((_D, 3 * _D), lambda i: (0, 0)),
            pl.BlockSpec((1, 3 * _D), lambda i: (0, 0)),
            pl.BlockSpec((_ROWBLK, _HD), lambda i: (i % (_T // _ROWBLK), 0)),
            pl.BlockSpec((_ROWBLK, _HD), lambda i: (i % (_T // _ROWBLK), 0)),
            pl.BlockSpec((_HD, _HD), lambda i: (0, 0)),
        ],
        out_specs=[
            pl.BlockSpec((_ROWBLK, _H * 256), lambda i: (i, 0)),
            pl.BlockSpec((_ROWBLK, _H), lambda i: (i, 0)),
        ],
        out_shape=[
            jax.ShapeDtypeStruct((n_rows, _H * 256), jnp.float32),
            jax.ShapeDtypeStruct((n_rows, _H), jnp.int32),
        ],
    )(x2, w_perm, b_perm, cos, sin, hash_w)


_LPC = 128
_NTC = _T // _LPC


def _count_sort_body(b_ref, tu128_ref, tu64_ref, tl128_ref, dest_ref):
    br = b_ref[0]
    bins = lax.broadcasted_iota(jnp.int32, (_NBUCKETS, _NTC, _LPC), 0)
    m3 = (br[None, :, :] == bins).astype(jnp.float32)
    r3 = lax.dot_general(m3, tu128_ref[...], (((2,), (0,)), ((), ())),
                         preferred_element_type=jnp.float32)
    g = jnp.sum(m3, axis=2)
    e = jnp.dot(g, tu64_ref[...],
                preferred_element_type=jnp.float32)
    cnt = jnp.sum(g, axis=1, keepdims=True)
    bo = jnp.dot(tl128_ref[...], cnt,
                 preferred_element_type=jnp.float32)
    dest3 = m3 * (r3 + e[:, :, None] + bo[:, :, None])
    dest_ref[0] = jnp.sum(dest3, axis=0).astype(jnp.int32)


def _count_sort(bidx3d):
    i128 = jnp.arange(_NBUCKETS)
    tu128 = (i128[:, None] < i128[None, :]).astype(jnp.float32)
    i64 = jnp.arange(_NTC)
    tu64 = (i64[:, None] < i64[None, :]).astype(jnp.float32)
    tl128 = (i128[None, :] < i128[:, None]).astype(jnp.float32)
    return pl.pallas_call(
        _count_sort_body,
        grid=(_B * _H,),
        in_specs=[
            pl.BlockSpec((1, _NTC, _LPC), lambda i: (i, 0, 0)),
            pl.BlockSpec((_NBUCKETS, _LPC), lambda i: (0, 0)),
            pl.BlockSpec((_NTC, _NTC), lambda i: (0, 0)),
            pl.BlockSpec((_NBUCKETS, _NBUCKETS), lambda i: (0, 0)),
        ],
        out_specs=pl.BlockSpec((1, _NTC, _LPC), lambda i: (i, 0, 0)),
        out_shape=jax.ShapeDtypeStruct((_B * _H, _NTC, _LPC), jnp.int32),
    )(bidx3d, tu128, tu64, tl128)


_ATTN_BATCH = 16


def _chunk_attn_body(s_ref, o_ref):
    s = s_ref[...]
    q = s[:, :, :_HD]
    k = s[:, :, _HD:2 * _HD]
    v = s[:, :, 2 * _HD:3 * _HD]
    a = lax.dot_general(q, k, (((2,), (2,)), ((0,), (0,))),
                        preferred_element_type=jnp.float32)
    a = a * (1.0 / math.sqrt(_HD))
    m = jnp.max(a, axis=-1, keepdims=True)
    e = jnp.exp(a - m)
    a = e / jnp.sum(e, axis=-1, keepdims=True)
    o = lax.dot_general(a, v, (((2,), (1,)), ((0,), (0,))),
                        preferred_element_type=jnp.float32)
    o_ref[...] = jnp.concatenate(
        [o, jnp.zeros(o.shape, jnp.float32)], axis=2)


def _chunk_attn(sorted_rows):
    n_blk = sorted_rows.shape[0]
    ab = _ATTN_BATCH
    return pl.pallas_call(
        _chunk_attn_body,
        grid=(n_blk // ab,),
        in_specs=[pl.BlockSpec((ab, _CHUNK, 4 * _HD), lambda i: (i, 0, 0))],
        out_specs=pl.BlockSpec((ab, _CHUNK, 2 * _HD), lambda i: (i, 0, 0)),
        out_shape=jax.ShapeDtypeStruct((n_blk, _CHUNK, 2 * _HD), jnp.float32),
    )(sorted_rows)


def _proj_body(x_ref, w_ref, b_ref, o_ref):
    o_ref[...] = jnp.dot(x_ref[...], w_ref[...],
                         preferred_element_type=jnp.float32) + b_ref[...]


def _proj(y, w, b2):
    n_rows = y.shape[0]
    d_in = y.shape[1]
    return pl.pallas_call(
        _proj_body,
        grid=(n_rows // _ROWBLK,),
        in_specs=[
            pl.BlockSpec((_ROWBLK, d_in), lambda i: (i, 0)),
            pl.BlockSpec((d_in, _D), lambda i: (0, 0)),
            pl.BlockSpec((1, _D), lambda i: (0, 0)),
        ],
        out_specs=pl.BlockSpec((_ROWBLK, _D), lambda i: (i, 0)),
        out_shape=jax.ShapeDtypeStruct((n_rows, _D), jnp.float32),
    )(y, w, b2)


def _gather_rows(table, idx, width):
    n_rows = idx.shape[0]
    info = plsc.get_sparse_core_info()
    nc, ns = info.num_cores, info.num_subcores
    nw = nc * ns
    rows_per_w = n_rows // nw
    n_chunk = rows_per_w // _GIN_CH
    idx3 = idx.reshape(nw, n_chunk, _GIN_CH)
    mesh = plsc.VectorSubcoreMesh(core_axis_name="c", subcore_axis_name="s")

    @functools.partial(
        pl.kernel, mesh=mesh,
        out_type=jax.ShapeDtypeStruct((n_rows, width), jnp.float32),
        scratch_types=[
            pltpu.VMEM((n_chunk, _GIN_CH), jnp.int32),
            pltpu.VMEM((_GIN_CH, width), jnp.float32),
            pltpu.SemaphoreType.DMA,
        ],
    )
    def k(table_hbm, idx_hbm, out_hbm, idx_v, rows_v, sem):
        wid = lax.axis_index("s") * nc + lax.axis_index("c")
        base = wid * rows_per_w
        pltpu.sync_copy(idx_hbm.at[wid], idx_v)

        def body(j, carry):
            pltpu.async_copy(table_hbm.at[idx_v.at[j]], rows_v, sem).wait()
            pltpu.sync_copy(rows_v,
                            out_hbm.at[pl.ds(base + j * _GIN_CH, _GIN_CH)])
            return carry

        lax.fori_loop(0, n_chunk, body, 0)

    return k(table, idx3)


def _scatter_rows(table, idx, width):
    n_rows = idx.shape[0]
    info = plsc.get_sparse_core_info()
    nc, ns = info.num_cores, info.num_subcores
    nw = nc * ns
    rows_per_w = n_rows // nw
    n_chunk = rows_per_w // _GIN_CH
    idx3 = idx.reshape(nw, n_chunk, _GIN_CH)
    mesh = plsc.VectorSubcoreMesh(core_axis_name="c", subcore_axis_name="s")

    @functools.partial(
        pl.kernel, mesh=mesh,
        out_type=jax.ShapeDtypeStruct((n_rows, width), jnp.float32),
        scratch_types=[
            pltpu.VMEM((n_chunk, _GIN_CH), jnp.int32),
            pltpu.VMEM((_GIN_CH, width), jnp.float32),
            pltpu.SemaphoreType.DMA,
        ],
    )
    def k(table_hbm, idx_hbm, out_hbm, idx_v, rows_v, sem):
        wid = lax.axis_index("s") * nc + lax.axis_index("c")
        base = wid * rows_per_w
        pltpu.sync_copy(idx_hbm.at[wid], idx_v)

        def body(j, carry):
            pltpu.sync_copy(table_hbm.at[pl.ds(base + j * _GIN_CH, _GIN_CH)],
                            rows_v)
            pltpu.async_copy(rows_v, out_hbm.at[idx_v.at[j]], sem).wait()
            return carry

        lax.fori_loop(0, n_chunk, body, 0)

    return k(table, idx3)


def kernel(x, Wqkv, bqkv, Wproj, bproj):
    w_perm = Wqkv.reshape(_D, 3, _H, _HD).transpose(0, 2, 1, 3).reshape(
        _D, 3 * _D)
    b_perm = bqkv.reshape(3, _H, _HD).transpose(1, 0, 2).reshape(1, 3 * _D)

    cos, sin = _rope_tables()
    R = jax.random.normal(jax.random.key(_SEED), (_HD, 2, _NBUCKETS // 2),
                          dtype=x.dtype)
    r0 = R[:, 0, :]

    x2 = x.reshape(_B * _T, _D)

    qkv_rows, bidx = _qkv_rope_hash(x2, w_perm, b_perm, cos, sin, r0)
    table = qkv_rows.reshape(_B * _T * _H, 4 * _HD)

    bidx3d = bidx.reshape(_B, _T, _H).transpose(0, 2, 1).reshape(
        _B * _H, _NTC, _LPC)
    dest = _count_sort(bidx3d).reshape(_B * _H, _T)
    src = jnp.arange(_B * _H, dtype=jnp.int32)[:, None] * _T + dest
    gidx = (src.reshape(_B, _H, _T).transpose(0, 2, 1)).reshape(-1)

    sorted_rows = _scatter_rows(table, gidx, 4 * _HD)
    sorted_blk = sorted_rows.reshape(_B * _H * _NCHUNKS, _CHUNK, 4 * _HD)

    out_local = _chunk_attn(sorted_blk).reshape(_B * _H * _T, 2 * _HD)

    out_rows = _gather_rows(out_local, gidx, 2 * _HD)
    y = out_rows.reshape(_B * _T, 2 * _D)

    w2 = jnp.pad(Wproj.reshape(_H, _HD, _D), ((0, 0), (0, _HD), (0, 0)))
    out = _proj(y, w2.reshape(2 * _D, _D), bproj.reshape(1, _D))
    return out.reshape(_B, _T, _D)


# speedup vs baseline: 3.5442x; 1.0470x over previous
import functools
import math

import jax
import jax.numpy as jnp
from jax import lax
from jax.experimental import pallas as pl
from jax.experimental.pallas import tpu as pltpu
from jax.experimental.pallas import tpu_sc as plsc

_B, _T, _D = 2, 8192, 768
_H = 12
_HD = _D // _H
_BUCKET = 64
_NBUCKETS = _T // _BUCKET
_CHUNK = 2 * _BUCKET
_NCHUNKS = _T // _CHUNK
_SEED = 1234

_ROWBLK = 512
_GIN_CH = 128


def _rope_tables():
    inv_freq = 1.0 / (10000.0 ** (jnp.arange(0, _HD, 2, dtype=jnp.float32) / _HD))
    t = jnp.arange(_T, dtype=jnp.float32)
    freqs = jnp.einsum('i,j->ij', t, inv_freq)
    emb = jnp.concatenate([freqs, freqs], axis=-1)
    return jnp.cos(emb), jnp.sin(emb)


def _qkv_rope_hash_body(x_ref, w_ref, b_ref, cos_ref, sin_ref, hw_ref,
                        out_ref, bidx_ref):
    acc = jnp.dot(x_ref[...], w_ref[...],
                  preferred_element_type=jnp.float32) + b_ref[...]
    cos = cos_ref[...]
    sin = sin_ref[...]
    ones = jnp.ones_like(cos)
    zeros = jnp.zeros_like(sin)
    cos_f = jnp.concatenate([cos, cos, ones] * _H, axis=1)
    sin_f = jnp.concatenate([sin, sin, zeros] * _H, axis=1)
    r_left = jnp.roll(acc, -1, axis=1)
    r_right = jnp.roll(acc, 1, axis=1)
    lane = lax.broadcasted_iota(jnp.int32, acc.shape, 1)
    rot = jnp.where(lane % 2 == 0, -r_left, r_right)
    roped = acc * cos_f + rot * sin_f
    pad = jnp.zeros((acc.shape[0], _HD), jnp.float32)
    pieces = []
    for h in range(_H):
        pieces.append(roped[:, h * 192:(h + 1) * 192])
        pieces.append(pad)
    out_ref[...] = jnp.concatenate(pieces, axis=1)
    r0 = hw_ref[...]
    cols = []
    for h in range(_H):
        ph = jnp.dot(roped[:, h * 192:h * 192 + _HD], r0,
                     preferred_element_type=jnp.float32)
        full = jnp.concatenate([ph, -ph], axis=1)
        cols.append(jnp.argmax(full, axis=1, keepdims=True).astype(jnp.int32))
    bidx_ref[...] = jnp.concatenate(cols, axis=1)


def _qkv_rope_hash(x2, w_perm, b_perm, cos, sin, hash_w):
    n_rows = x2.shape[0]
    grid = (n_rows // _ROWBLK,)
    return pl.pallas_call(
        _qkv_rope_hash_body,
        grid=grid,
        in_specs=[
            pl.BlockSpec((_ROWBLK, _D), lambda i: (i, 0)),
            pl.BlockSpec((_D, 3 * _D), lambda i: (0, 0)),
            pl.BlockSpec((1, 3 * _D), lambda i: (0, 0)),
            pl.BlockSpec((_ROWBLK, _HD), lambda i: (i % (_T // _ROWBLK), 0)),
            pl.BlockSpec((_ROWBLK, _HD), lambda i: (i % (_T // _ROWBLK), 0)),
            pl.BlockSpec((_HD, _HD), lambda i: (0, 0)),
        ],
        out_specs=[
            pl.BlockSpec((_ROWBLK, _H * 256), lambda i: (i, 0)),
            pl.BlockSpec((_ROWBLK, _H), lambda i: (i, 0)),
        ],
        out_shape=[
            jax.ShapeDtypeStruct((n_rows, _H * 256), jnp.float32),
            jax.ShapeDtypeStruct((n_rows, _H), jnp.int32),
        ],
    )(x2, w_perm, b_perm, cos, sin, hash_w)


_LPC = 128
_NTC = _T // _LPC


def _count_sort_body(b_ref, tu128_ref, tu64_ref, tl128_ref, dest_ref):
    br = b_ref[0]
    bins = lax.broadcasted_iota(jnp.int32, (_NBUCKETS, _NTC, _LPC), 0)
    m3 = (br[None, :, :] == bins).astype(jnp.float32)
    r3 = lax.dot_general(m3, tu128_ref[...], (((2,), (0,)), ((), ())),
                         preferred_element_type=jnp.float32)
    g = jnp.sum(m3, axis=2)
    e = jnp.dot(g, tu64_ref[...],
                preferred_element_type=jnp.float32)
    cnt = jnp.sum(g, axis=1, keepdims=True)
    bo = jnp.dot(tl128_ref[...], cnt,
                 preferred_element_type=jnp.float32)
    dest3 = m3 * (r3 + e[:, :, None] + bo[:, :, None])
    dest_ref[0] = jnp.sum(dest3, axis=0).astype(jnp.int32)


def _count_sort(bidx3d):
    i128 = jnp.arange(_NBUCKETS)
    tu128 = (i128[:, None] < i128[None, :]).astype(jnp.float32)
    i64 = jnp.arange(_NTC)
    tu64 = (i64[:, None] < i64[None, :]).astype(jnp.float32)
    tl128 = (i128[None, :] < i128[:, None]).astype(jnp.float32)
    return pl.pallas_call(
        _count_sort_body,
        grid=(_B * _H,),
        in_specs=[
            pl.BlockSpec((1, _NTC, _LPC), lambda i: (i, 0, 0)),
            pl.BlockSpec((_NBUCKETS, _LPC), lambda i: (0, 0)),
            pl.BlockSpec((_NTC, _NTC), lambda i: (0, 0)),
            pl.BlockSpec((_NBUCKETS, _NBUCKETS), lambda i: (0, 0)),
        ],
        out_specs=pl.BlockSpec((1, _NTC, _LPC), lambda i: (i, 0, 0)),
        out_shape=jax.ShapeDtypeStruct((_B * _H, _NTC, _LPC), jnp.int32),
    )(bidx3d, tu128, tu64, tl128)


_ATTN_BATCH = 16


def _chunk_attn_body(s_ref, o_ref):
    s = s_ref[...]
    q = s[:, :, :_HD]
    k = s[:, :, _HD:2 * _HD]
    v = s[:, :, 2 * _HD:3 * _HD]
    a = lax.dot_general(q, k, (((2,), (2,)), ((0,), (0,))),
                        preferred_element_type=jnp.float32)
    a = a * (1.0 / math.sqrt(_HD))
    m = jnp.max(a, axis=-1, keepdims=True)
    e = jnp.exp(a - m)
    a = e / jnp.sum(e, axis=-1, keepdims=True)
    o = lax.dot_general(a, v, (((2,), (1,)), ((0,), (0,))),
                        preferred_element_type=jnp.float32)
    o_ref[...] = jnp.concatenate(
        [o, jnp.zeros(o.shape, jnp.float32)], axis=2)


def _chunk_attn(sorted_rows):
    n_blk = sorted_rows.shape[0]
    ab = _ATTN_BATCH
    return pl.pallas_call(
        _chunk_attn_body,
        grid=(n_blk // ab,),
        in_specs=[pl.BlockSpec((ab, _CHUNK, 4 * _HD), lambda i: (i, 0, 0))],
        out_specs=pl.BlockSpec((ab, _CHUNK, 2 * _HD), lambda i: (i, 0, 0)),
        out_shape=jax.ShapeDtypeStruct((n_blk, _CHUNK, 2 * _HD), jnp.float32),
    )(sorted_rows)


def _proj_body(x_ref, w_ref, b_ref, o_ref):
    o_ref[...] = jnp.dot(x_ref[...], w_ref[...],
                         preferred_element_type=jnp.float32) + b_ref[...]


def _proj(y, w, b2):
    n_rows = y.shape[0]
    d_in = y.shape[1]
    return pl.pallas_call(
        _proj_body,
        grid=(n_rows // _ROWBLK,),
        in_specs=[
            pl.BlockSpec((_ROWBLK, d_in), lambda i: (i, 0)),
            pl.BlockSpec((d_in, _D), lambda i: (0, 0)),
            pl.BlockSpec((1, _D), lambda i: (0, 0)),
        ],
        out_specs=pl.BlockSpec((_ROWBLK, _D), lambda i: (i, 0)),
        out_shape=jax.ShapeDtypeStruct((n_rows, _D), jnp.float32),
    )(y, w, b2)


def _shuffle_rows(table, idx, width, scatter):
    n_rows = idx.shape[0]
    info = plsc.get_sparse_core_info()
    nc, ns = info.num_cores, info.num_subcores
    nw = nc * ns
    rows_per_w = n_rows // nw
    n_chunk = rows_per_w // _GIN_CH
    idx3 = idx.reshape(nw, n_chunk, _GIN_CH)
    mesh = plsc.VectorSubcoreMesh(core_axis_name="c", subcore_axis_name="s")

    @functools.partial(
        pl.kernel, mesh=mesh,
        out_type=jax.ShapeDtypeStruct((n_rows, width), jnp.float32),
        scratch_types=[
            pltpu.VMEM((n_chunk, _GIN_CH), jnp.int32),
            pltpu.VMEM((_GIN_CH, width), jnp.float32),
            pltpu.VMEM((_GIN_CH, width), jnp.float32),
            pltpu.SemaphoreType.DMA,
            pltpu.SemaphoreType.DMA,
            pltpu.SemaphoreType.DMA,
            pltpu.SemaphoreType.DMA,
        ],
    )
    def k(table_hbm, idx_hbm, out_hbm, idx_v, buf0, buf1,
          r0, r1, w0, w1):
        wid = lax.axis_index("s") * nc + lax.axis_index("c")
        base = wid * rows_per_w
        pltpu.sync_copy(idx_hbm.at[wid], idx_v)

        def legs(j, buf, rsem, wsem):
            if scatter:
                rd = pltpu.async_copy(table_hbm.at[pl.ds(base + j * _GIN_CH,
                                                         _GIN_CH)], buf, rsem)
                return rd, lambda: pltpu.async_copy(
                    buf, out_hbm.at[idx_v.at[j]], wsem)
            rd = pltpu.async_copy(table_hbm.at[idx_v.at[j]], buf, rsem)
            return rd, lambda: pltpu.async_copy(
                buf, out_hbm.at[pl.ds(base + j * _GIN_CH, _GIN_CH)], wsem)

        def body(jj, carry):
            j0 = jj * 2
            j1 = j0 + 1
            rd0, wr0 = legs(j0, buf0, r0, w0)
            rd1, wr1 = legs(j1, buf1, r1, w1)
            rd0.wait()
            h0 = wr0()
            rd1.wait()
            h1 = wr1()
            h0.wait()
            h1.wait()
            return carry

        lax.fori_loop(0, n_chunk // 2, body, 0)

    return k(table, idx3)


def _gather_rows(table, idx, width):
    return _shuffle_rows(table, idx, width, scatter=False)


def _scatter_rows(table, idx, width):
    return _shuffle_rows(table, idx, width, scatter=True)


def kernel(x, Wqkv, bqkv, Wproj, bproj):
    w_perm = Wqkv.reshape(_D, 3, _H, _HD).transpose(0, 2, 1, 3).reshape(
        _D, 3 * _D)
    b_perm = bqkv.reshape(3, _H, _HD).transpose(1, 0, 2).reshape(1, 3 * _D)

    cos, sin = _rope_tables()
    R = jax.random.normal(jax.random.key(_SEED), (_HD, 2, _NBUCKETS // 2),
                          dtype=x.dtype)
    r0 = R[:, 0, :]

    x2 = x.reshape(_B * _T, _D)

    qkv_rows, bidx = _qkv_rope_hash(x2, w_perm, b_perm, cos, sin, r0)
    table = qkv_rows.reshape(_B * _T * _H, 4 * _HD)

    bidx3d = bidx.reshape(_B, _T, _H).transpose(0, 2, 1).reshape(
        _B * _H, _NTC, _LPC)
    dest = _count_sort(bidx3d).reshape(_B * _H, _T)
    src = jnp.arange(_B * _H, dtype=jnp.int32)[:, None] * _T + dest
    gidx = (src.reshape(_B, _H, _T).transpose(0, 2, 1)).reshape(-1)

    sorted_rows = _scatter_rows(table, gidx, 4 * _HD)
    sorted_blk = sorted_rows.reshape(_B * _H * _NCHUNKS, _CHUNK, 4 * _HD)

    out_local = _chunk_attn(sorted_blk).reshape(_B * _H * _T, 2 * _HD)

    out_rows = _gather_rows(out_local, gidx, 2 * _HD)
    y = out_rows.reshape(_B * _T, 2 * _D)

    w2 = jnp.pad(Wproj.reshape(_H, _HD, _D), ((0, 0), (0, _HD), (0, 0)))
    out = _proj(y, w2.reshape(2 * _D, _D), bproj.reshape(1, _D))
    return out.reshape(_B, _T, _D)


# speedup vs baseline: 3.5971x; 1.0149x over previous
import functools
import math

import jax
import jax.numpy as jnp
from jax import lax
from jax.experimental import pallas as pl
from jax.experimental.pallas import tpu as pltpu
from jax.experimental.pallas import tpu_sc as plsc

_B, _T, _D = 2, 8192, 768
_H = 12
_HD = _D // _H
_BUCKET = 64
_NBUCKETS = _T // _BUCKET
_CHUNK = 2 * _BUCKET
_NCHUNKS = _T // _CHUNK
_SEED = 1234

_ROWBLK = 512
_GIN_CH = 128


def _rope_tables():
    inv_freq = 1.0 / (10000.0 ** (jnp.arange(0, _HD, 2, dtype=jnp.float32) / _HD))
    t = jnp.arange(_T, dtype=jnp.float32)
    freqs = jnp.einsum('i,j->ij', t, inv_freq)
    emb = jnp.concatenate([freqs, freqs], axis=-1)
    return jnp.cos(emb), jnp.sin(emb)


def _qkv_rope_hash_body(x_ref, w_ref, b_ref, cos_ref, sin_ref, hw_ref,
                        out_ref, bidx_ref):
    acc = jnp.dot(x_ref[...], w_ref[...],
                  preferred_element_type=jnp.float32) + b_ref[...]
    cos = cos_ref[...]
    sin = sin_ref[...]
    ones = jnp.ones_like(cos)
    zeros = jnp.zeros_like(sin)
    cos_f = jnp.concatenate([cos, cos, ones] * _H, axis=1)
    sin_f = jnp.concatenate([sin, sin, zeros] * _H, axis=1)
    r_left = jnp.roll(acc, -1, axis=1)
    r_right = jnp.roll(acc, 1, axis=1)
    lane = lax.broadcasted_iota(jnp.int32, acc.shape, 1)
    rot = jnp.where(lane % 2 == 0, -r_left, r_right)
    roped = acc * cos_f + rot * sin_f
    pad = jnp.zeros((acc.shape[0], _HD), jnp.float32)
    pieces = []
    for h in range(_H):
        pieces.append(roped[:, h * 192:(h + 1) * 192])
        pieces.append(pad)
    out_ref[...] = jnp.concatenate(pieces, axis=1)
    r0 = hw_ref[...]
    cols = []
    for h in range(_H):
        ph = jnp.dot(roped[:, h * 192:h * 192 + _HD], r0,
                     preferred_element_type=jnp.float32)
        full = jnp.concatenate([ph, -ph], axis=1)
        cols.append(jnp.argmax(full, axis=1, keepdims=True).astype(jnp.int32))
    bidx_ref[...] = jnp.concatenate(cols, axis=1)


def _qkv_rope_hash(x2, w_perm, b_perm, cos, sin, hash_w):
    n_rows = x2.shape[0]
    grid = (n_rows // _ROWBLK,)
    return pl.pallas_call(
        _qkv_rope_hash_body,
        grid=grid,
        in_specs=[
            pl.BlockSpec((_ROWBLK, _D), lambda i: (i, 0)),
            pl.BlockSpec((_D, 3 * _D), lambda i: (0, 0)),
            pl.BlockSpec((1, 3 * _D), lambda i: (0, 0)),
            pl.BlockSpec((_ROWBLK, _HD), lambda i: (i % (_T // _ROWBLK), 0)),
            pl.BlockSpec((_ROWBLK, _HD), lambda i: (i % (_T // _ROWBLK), 0)),
            pl.BlockSpec((_HD, _HD), lambda i: (0, 0)),
        ],
        out_specs=[
            pl.BlockSpec((_ROWBLK, _H * 256), lambda i: (i, 0)),
            pl.BlockSpec((_ROWBLK, _H), lambda i: (i, 0)),
        ],
        out_shape=[
            jax.ShapeDtypeStruct((n_rows, _H * 256), jnp.float32),
            jax.ShapeDtypeStruct((n_rows, _H), jnp.int32),
        ],
    )(x2, w_perm, b_perm, cos, sin, hash_w)


_LPC = 128
_NTC = _T // _LPC


def _count_sort_body(b_ref, tu128_ref, tu64_ref, tl128_ref, dest_ref):
    br = b_ref[0]
    bins = lax.broadcasted_iota(jnp.int32, (_NBUCKETS, _NTC, _LPC), 0)
    m3 = (br[None, :, :] == bins).astype(jnp.float32)
    r3 = lax.dot_general(m3, tu128_ref[...], (((2,), (0,)), ((), ())),
                         preferred_element_type=jnp.float32)
    g = jnp.sum(m3, axis=2)
    e = jnp.dot(g, tu64_ref[...],
                preferred_element_type=jnp.float32)
    cnt = jnp.sum(g, axis=1, keepdims=True)
    bo = jnp.dot(tl128_ref[...], cnt,
                 preferred_element_type=jnp.float32)
    dest3 = m3 * (r3 + e[:, :, None] + bo[:, :, None])
    dest_ref[0] = jnp.sum(dest3, axis=0).astype(jnp.int32)


def _count_sort(bidx3d):
    i128 = jnp.arange(_NBUCKETS)
    tu128 = (i128[:, None] < i128[None, :]).astype(jnp.float32)
    i64 = jnp.arange(_NTC)
    tu64 = (i64[:, None] < i64[None, :]).astype(jnp.float32)
    tl128 = (i128[None, :] < i128[:, None]).astype(jnp.float32)
    n_rows = bidx3d.shape[0]
    return pl.pallas_call(
        _count_sort_body,
        grid=(n_rows,),
        in_specs=[
            pl.BlockSpec((1, _NTC, _LPC), lambda i: (i, 0, 0)),
            pl.BlockSpec((_NBUCKETS, _LPC), lambda i: (0, 0)),
            pl.BlockSpec((_NTC, _NTC), lambda i: (0, 0)),
            pl.BlockSpec((_NBUCKETS, _NBUCKETS), lambda i: (0, 0)),
        ],
        out_specs=pl.BlockSpec((1, _NTC, _LPC), lambda i: (i, 0, 0)),
        out_shape=jax.ShapeDtypeStruct((n_rows, _NTC, _LPC), jnp.int32),
    )(bidx3d, tu128, tu64, tl128)


_ATTN_BATCH = 16


def _chunk_attn_body(s_ref, o_ref):
    s = s_ref[...]
    q = s[:, :, :_HD]
    k = s[:, :, _HD:2 * _HD]
    v = s[:, :, 2 * _HD:3 * _HD]
    a = lax.dot_general(q, k, (((2,), (2,)), ((0,), (0,))),
                        preferred_element_type=jnp.float32)
    a = a * (1.0 / math.sqrt(_HD))
    m = jnp.max(a, axis=-1, keepdims=True)
    e = jnp.exp(a - m)
    a = e / jnp.sum(e, axis=-1, keepdims=True)
    o = lax.dot_general(a, v, (((2,), (1,)), ((0,), (0,))),
                        preferred_element_type=jnp.float32)
    o_ref[...] = jnp.concatenate(
        [o, jnp.zeros(o.shape, jnp.float32)], axis=2)


def _chunk_attn(sorted_rows):
    n_blk = sorted_rows.shape[0]
    ab = _ATTN_BATCH
    return pl.pallas_call(
        _chunk_attn_body,
        grid=(n_blk // ab,),
        in_specs=[pl.BlockSpec((ab, _CHUNK, 4 * _HD), lambda i: (i, 0, 0))],
        out_specs=pl.BlockSpec((ab, _CHUNK, 2 * _HD), lambda i: (i, 0, 0)),
        out_shape=jax.ShapeDtypeStruct((n_blk, _CHUNK, 2 * _HD), jnp.float32),
    )(sorted_rows)


def _proj_body(x_ref, w_ref, b_ref, o_ref):
    o_ref[...] = jnp.dot(x_ref[...], w_ref[...],
                         preferred_element_type=jnp.float32) + b_ref[...]


def _proj(y, w, b2):
    n_rows = y.shape[0]
    d_in = y.shape[1]
    return pl.pallas_call(
        _proj_body,
        grid=(n_rows // _ROWBLK,),
        in_specs=[
            pl.BlockSpec((_ROWBLK, d_in), lambda i: (i, 0)),
            pl.BlockSpec((d_in, _D), lambda i: (0, 0)),
            pl.BlockSpec((1, _D), lambda i: (0, 0)),
        ],
        out_specs=pl.BlockSpec((_ROWBLK, _D), lambda i: (i, 0)),
        out_shape=jax.ShapeDtypeStruct((n_rows, _D), jnp.float32),
    )(y, w, b2)


def _shuffle_rows(table, idx, width, scatter):
    n_rows = idx.shape[0]
    info = plsc.get_sparse_core_info()
    nc, ns = info.num_cores, info.num_subcores
    nw = nc * ns
    rows_per_w = n_rows // nw
    n_chunk = rows_per_w // _GIN_CH
    idx3 = idx.reshape(nw, n_chunk, _GIN_CH)
    mesh = plsc.VectorSubcoreMesh(core_axis_name="c", subcore_axis_name="s")

    @functools.partial(
        pl.kernel, mesh=mesh,
        out_type=jax.ShapeDtypeStruct((n_rows, width), jnp.float32),
        scratch_types=[
            pltpu.VMEM((n_chunk, _GIN_CH), jnp.int32),
            pltpu.VMEM((_GIN_CH, width), jnp.float32),
            pltpu.VMEM((_GIN_CH, width), jnp.float32),
            pltpu.SemaphoreType.DMA,
            pltpu.SemaphoreType.DMA,
            pltpu.SemaphoreType.DMA,
            pltpu.SemaphoreType.DMA,
        ],
    )
    def k(table_hbm, idx_hbm, out_hbm, idx_v, buf0, buf1,
          r0, r1, w0, w1):
        wid = lax.axis_index("s") * nc + lax.axis_index("c")
        base = wid * rows_per_w
        pltpu.sync_copy(idx_hbm.at[wid], idx_v)

        def legs(j, buf, rsem, wsem):
            if scatter:
                rd = pltpu.async_copy(table_hbm.at[pl.ds(base + j * _GIN_CH,
                                                         _GIN_CH)], buf, rsem)
                return rd, lambda: pltpu.async_copy(
                    buf, out_hbm.at[idx_v.at[j]], wsem)
            rd = pltpu.async_copy(table_hbm.at[idx_v.at[j]], buf, rsem)
            return rd, lambda: pltpu.async_copy(
                buf, out_hbm.at[pl.ds(base + j * _GIN_CH, _GIN_CH)], wsem)

        def body(jj, carry):
            j0 = jj * 2
            j1 = j0 + 1
            rd0, wr0 = legs(j0, buf0, r0, w0)
            rd1, wr1 = legs(j1, buf1, r1, w1)
            rd0.wait()
            h0 = wr0()
            rd1.wait()
            h1 = wr1()
            h0.wait()
            h1.wait()
            return carry

        lax.fori_loop(0, n_chunk // 2, body, 0)

    return k(table, idx3)


def _gather_rows(table, idx, width):
    return _shuffle_rows(table, idx, width, scatter=False)


def _scatter_rows(table, idx, width):
    return _shuffle_rows(table, idx, width, scatter=True)


def kernel(x, Wqkv, bqkv, Wproj, bproj):
    w_perm = Wqkv.reshape(_D, 3, _H, _HD).transpose(0, 2, 1, 3).reshape(
        _D, 3 * _D)
    b_perm = bqkv.reshape(3, _H, _HD).transpose(1, 0, 2).reshape(1, 3 * _D)

    cos, sin = _rope_tables()
    R = jax.random.normal(jax.random.key(_SEED), (_HD, 2, _NBUCKETS // 2),
                          dtype=x.dtype)
    r0 = R[:, 0, :]

    w2 = jnp.pad(Wproj.reshape(_H, _HD, _D), ((0, 0), (0, _HD), (0, 0)))
    w2 = w2.reshape(2 * _D, _D)
    b2 = bproj.reshape(1, _D)

    outs = []
    for b in range(_B):
        x2 = x[b].reshape(_T, _D)

        qkv_rows, bidx = _qkv_rope_hash(x2, w_perm, b_perm, cos, sin, r0)
        table = qkv_rows.reshape(_T * _H, 4 * _HD)

        bidx3d = bidx.reshape(_T, _H).transpose(1, 0).reshape(
            _H, _NTC, _LPC)
        dest = _count_sort(bidx3d).reshape(_H, _T)
        src = jnp.arange(_H, dtype=jnp.int32)[:, None] * _T + dest
        gidx = src.transpose(1, 0).reshape(-1)

        sorted_rows = _scatter_rows(table, gidx, 4 * _HD)
        sorted_blk = sorted_rows.reshape(_H * _NCHUNKS, _CHUNK, 4 * _HD)

        out_local = _chunk_attn(sorted_blk).reshape(_H * _T, 2 * _HD)

        out_rows = _gather_rows(out_local, gidx, 2 * _HD)
        y = out_rows.reshape(_T, 2 * _D)

        outs.append(_proj(y, w2, b2))

    return jnp.stack(outs, axis=0)


# speedup vs baseline: 3.6445x; 1.0132x over previous
import functools
import math

import jax
import jax.numpy as jnp
from jax import lax
from jax.experimental import pallas as pl
from jax.experimental.pallas import tpu as pltpu
from jax.experimental.pallas import tpu_sc as plsc

_B, _T, _D = 2, 8192, 768
_H = 12
_HD = _D // _H
_BUCKET = 64
_NBUCKETS = _T // _BUCKET
_CHUNK = 2 * _BUCKET
_NCHUNKS = _T // _CHUNK
_SEED = 1234

_ROWBLK = 512
_GIN_CH = 128


def _rope_tables():
    inv_freq = 1.0 / (10000.0 ** (jnp.arange(0, _HD, 2, dtype=jnp.float32) / _HD))
    t = jnp.arange(_T, dtype=jnp.float32)
    freqs = jnp.einsum('i,j->ij', t, inv_freq)
    emb = jnp.concatenate([freqs, freqs], axis=-1)
    return jnp.cos(emb), jnp.sin(emb)


def _qkv_rope_hash_body(x_ref, w_ref, b_ref, cos_ref, sin_ref, hw_ref,
                        out_ref, bidx_ref):
    acc = jnp.dot(x_ref[...], w_ref[...],
                  preferred_element_type=jnp.float32) + b_ref[...]
    cos = cos_ref[...]
    sin = sin_ref[...]
    ones = jnp.ones_like(cos)
    zeros = jnp.zeros_like(sin)
    cos_f = jnp.concatenate([cos, cos, ones] * _H, axis=1)
    sin_f = jnp.concatenate([sin, sin, zeros] * _H, axis=1)
    r_left = jnp.roll(acc, -1, axis=1)
    r_right = jnp.roll(acc, 1, axis=1)
    lane = lax.broadcasted_iota(jnp.int32, acc.shape, 1)
    rot = jnp.where(lane % 2 == 0, -r_left, r_right)
    roped = acc * cos_f + rot * sin_f
    pad = jnp.zeros((acc.shape[0], _HD), jnp.float32)
    pieces = []
    for h in range(_H):
        pieces.append(roped[:, h * 192:(h + 1) * 192])
        pieces.append(pad)
    out_ref[...] = jnp.concatenate(pieces, axis=1)
    r0 = hw_ref[...]
    cols = []
    for h in range(_H):
        ph = jnp.dot(roped[:, h * 192:h * 192 + _HD], r0,
                     preferred_element_type=jnp.float32)
        full = jnp.concatenate([ph, -ph], axis=1)
        cols.append(jnp.argmax(full, axis=1, keepdims=True).astype(jnp.int32))
    bidx_ref[...] = jnp.concatenate(cols, axis=1)


def _qkv_rope_hash(x2, w_perm, b_perm, cos, sin, hash_w):
    n_rows = x2.shape[0]
    grid = (n_rows // _ROWBLK,)
    return pl.pallas_call(
        _qkv_rope_hash_body,
        grid=grid,
        in_specs=[
            pl.BlockSpec((_ROWBLK, _D), lambda i: (i, 0)),
            pl.BlockSpec((_D, 3 * _D), lambda i: (0, 0)),
            pl.BlockSpec((1, 3 * _D), lambda i: (0, 0)),
            pl.BlockSpec((_ROWBLK, _HD), lambda i: (i % (_T // _ROWBLK), 0)),
            pl.BlockSpec((_ROWBLK, _HD), lambda i: (i % (_T // _ROWBLK), 0)),
            pl.BlockSpec((_HD, _HD), lambda i: (0, 0)),
        ],
        out_specs=[
            pl.BlockSpec((_ROWBLK, _H * 256), lambda i: (i, 0)),
            pl.BlockSpec((_ROWBLK, _H), lambda i: (i, 0)),
        ],
        out_shape=[
            jax.ShapeDtypeStruct((n_rows, _H * 256), jnp.float32),
            jax.ShapeDtypeStruct((n_rows, _H), jnp.int32),
        ],
    )(x2, w_perm, b_perm, cos, sin, hash_w)


_LPC = 128
_NTC = _T // _LPC


def _count_sort_body(b_ref, tu128_ref, tu64_ref, tl128_ref, dest_ref):
    br = b_ref[0]
    bins = lax.broadcasted_iota(jnp.int32, (_NBUCKETS, _NTC, _LPC), 0)
    m3 = (br[None, :, :] == bins).astype(jnp.float32)
    r3 = lax.dot_general(m3, tu128_ref[...], (((2,), (0,)), ((), ())),
                         preferred_element_type=jnp.float32)
    g = jnp.sum(m3, axis=2)
    e = jnp.dot(g, tu64_ref[...],
                preferred_element_type=jnp.float32)
    cnt = jnp.sum(g, axis=1, keepdims=True)
    bo = jnp.dot(tl128_ref[...], cnt,
                 preferred_element_type=jnp.float32)
    dest3 = m3 * (r3 + e[:, :, None] + bo[:, :, None])
    dest_ref[0] = jnp.sum(dest3, axis=0).astype(jnp.int32)


def _count_sort(bidx3d):
    i128 = jnp.arange(_NBUCKETS)
    tu128 = (i128[:, None] < i128[None, :]).astype(jnp.float32)
    i64 = jnp.arange(_NTC)
    tu64 = (i64[:, None] < i64[None, :]).astype(jnp.float32)
    tl128 = (i128[None, :] < i128[:, None]).astype(jnp.float32)
    n_rows = bidx3d.shape[0]
    return pl.pallas_call(
        _count_sort_body,
        grid=(n_rows,),
        in_specs=[
            pl.BlockSpec((1, _NTC, _LPC), lambda i: (i, 0, 0)),
            pl.BlockSpec((_NBUCKETS, _LPC), lambda i: (0, 0)),
            pl.BlockSpec((_NTC, _NTC), lambda i: (0, 0)),
            pl.BlockSpec((_NBUCKETS, _NBUCKETS), lambda i: (0, 0)),
        ],
        out_specs=pl.BlockSpec((1, _NTC, _LPC), lambda i: (i, 0, 0)),
        out_shape=jax.ShapeDtypeStruct((n_rows, _NTC, _LPC), jnp.int32),
    )(bidx3d, tu128, tu64, tl128)


_ATTN_BATCH = 32


def _chunk_attn_body(s_ref, o_ref):
    s = s_ref[...]
    q = s[:, :, :_HD]
    k = s[:, :, _HD:2 * _HD]
    v = s[:, :, 2 * _HD:3 * _HD]
    a = lax.dot_general(q, k, (((2,), (2,)), ((0,), (0,))),
                        preferred_element_type=jnp.float32)
    a = a * (1.0 / math.sqrt(_HD))
    m = jnp.max(a, axis=-1, keepdims=True)
    e = jnp.exp(a - m)
    a = e / jnp.sum(e, axis=-1, keepdims=True)
    o = lax.dot_general(a, v, (((2,), (1,)), ((0,), (0,))),
                        preferred_element_type=jnp.float32)
    o_ref[...] = jnp.concatenate(
        [o, jnp.zeros(o.shape, jnp.float32)], axis=2)


def _chunk_attn(sorted_rows):
    n_blk = sorted_rows.shape[0]
    ab = _ATTN_BATCH
    return pl.pallas_call(
        _chunk_attn_body,
        grid=(n_blk // ab,),
        in_specs=[pl.BlockSpec((ab, _CHUNK, 4 * _HD), lambda i: (i, 0, 0))],
        out_specs=pl.BlockSpec((ab, _CHUNK, 2 * _HD), lambda i: (i, 0, 0)),
        out_shape=jax.ShapeDtypeStruct((n_blk, _CHUNK, 2 * _HD), jnp.float32),
    )(sorted_rows)


def _proj_body(x_ref, w_ref, b_ref, o_ref):
    o_ref[...] = jnp.dot(x_ref[...], w_ref[...],
                         preferred_element_type=jnp.float32) + b_ref[...]


def _proj(y, w, b2):
    n_rows = y.shape[0]
    d_in = y.shape[1]
    return pl.pallas_call(
        _proj_body,
        grid=(n_rows // _ROWBLK,),
        in_specs=[
            pl.BlockSpec((_ROWBLK, d_in), lambda i: (i, 0)),
            pl.BlockSpec((d_in, _D), lambda i: (0, 0)),
            pl.BlockSpec((1, _D), lambda i: (0, 0)),
        ],
        out_specs=pl.BlockSpec((_ROWBLK, _D), lambda i: (i, 0)),
        out_shape=jax.ShapeDtypeStruct((n_rows, _D), jnp.float32),
    )(y, w, b2)


def _shuffle_rows(table, idx, width, scatter):
    n_rows = idx.shape[0]
    info = plsc.get_sparse_core_info()
    nc, ns = info.num_cores, info.num_subcores
    nw = nc * ns
    rows_per_w = n_rows // nw
    n_chunk = rows_per_w // _GIN_CH
    idx3 = idx.reshape(nw, n_chunk, _GIN_CH)
    mesh = plsc.VectorSubcoreMesh(core_axis_name="c", subcore_axis_name="s")

    @functools.partial(
        pl.kernel, mesh=mesh,
        out_type=jax.ShapeDtypeStruct((n_rows, width), jnp.float32),
        scratch_types=[
            pltpu.VMEM((n_chunk, _GIN_CH), jnp.int32),
            pltpu.VMEM((_GIN_CH, width), jnp.float32),
            pltpu.VMEM((_GIN_CH, width), jnp.float32),
            pltpu.SemaphoreType.DMA,
            pltpu.SemaphoreType.DMA,
            pltpu.SemaphoreType.DMA,
            pltpu.SemaphoreType.DMA,
        ],
    )
    def k(table_hbm, idx_hbm, out_hbm, idx_v, buf0, buf1,
          r0, r1, w0, w1):
        wid = lax.axis_index("s") * nc + lax.axis_index("c")
        base = wid * rows_per_w
        pltpu.sync_copy(idx_hbm.at[wid], idx_v)

        def legs(j, buf, rsem, wsem):
            if scatter:
                rd = pltpu.async_copy(table_hbm.at[pl.ds(base + j * _GIN_CH,
                                                         _GIN_CH)], buf, rsem)
                return rd, lambda: pltpu.async_copy(
                    buf, out_hbm.at[idx_v.at[j]], wsem)
            rd = pltpu.async_copy(table_hbm.at[idx_v.at[j]], buf, rsem)
            return rd, lambda: pltpu.async_copy(
                buf, out_hbm.at[pl.ds(base + j * _GIN_CH, _GIN_CH)], wsem)

        def body(jj, carry):
            j0 = jj * 2
            j1 = j0 + 1
            rd0, wr0 = legs(j0, buf0, r0, w0)
            rd1, wr1 = legs(j1, buf1, r1, w1)
            rd0.wait()
            h0 = wr0()
            rd1.wait()
            h1 = wr1()
            h0.wait()
            h1.wait()
            return carry

        lax.fori_loop(0, n_chunk // 2, body, 0)

    return k(table, idx3)


def _gather_rows(table, idx, width):
    return _shuffle_rows(table, idx, width, scatter=False)


def _scatter_rows(table, idx, width):
    return _shuffle_rows(table, idx, width, scatter=True)


def kernel(x, Wqkv, bqkv, Wproj, bproj):
    w_perm = Wqkv.reshape(_D, 3, _H, _HD).transpose(0, 2, 1, 3).reshape(
        _D, 3 * _D)
    b_perm = bqkv.reshape(3, _H, _HD).transpose(1, 0, 2).reshape(1, 3 * _D)

    cos, sin = _rope_tables()
    R = jax.random.normal(jax.random.key(_SEED), (_HD, 2, _NBUCKETS // 2),
                          dtype=x.dtype)
    r0 = R[:, 0, :]

    w2 = jnp.pad(Wproj.reshape(_H, _HD, _D), ((0, 0), (0, _HD), (0, 0)))
    w2 = w2.reshape(2 * _D, _D)
    b2 = bproj.reshape(1, _D)

    outs = []
    for b in range(_B):
        x2 = x[b].reshape(_T, _D)

        qkv_rows, bidx = _qkv_rope_hash(x2, w_perm, b_perm, cos, sin, r0)
        table = qkv_rows.reshape(_T * _H, 4 * _HD)

        bidx3d = bidx.reshape(_T, _H).transpose(1, 0).reshape(
            _H, _NTC, _LPC)
        dest = _count_sort(bidx3d).reshape(_H, _T)
        src = jnp.arange(_H, dtype=jnp.int32)[:, None] * _T + dest
        gidx = src.transpose(1, 0).reshape(-1)

        sorted_rows = _scatter_rows(table, gidx, 4 * _HD)
        sorted_blk = sorted_rows.reshape(_H * _NCHUNKS, _CHUNK, 4 * _HD)

        out_local = _chunk_attn(sorted_blk).reshape(_H * _T, 2 * _HD)

        out_rows = _gather_rows(out_local, gidx, 2 * _HD)
        y = out_rows.reshape(_T, 2 * _D)

        outs.append(_proj(y, w2, b2))

    return jnp.stack(outs, axis=0)
